# initial kernel scaffold (unmeasured)
import jax
import jax.numpy as jnp
from jax import lax
from jax.experimental import pallas as pl
from jax.experimental.pallas import tpu as pltpu

N_DEV = 8
SQ = 256
HQ = 8
DH = 128
NJ = 4
KSEL = 1024
SCALE = 0.08838834764831843


def _body(q_ref, kt_ref, v_ref, wo_ref, out_ref,
          qc_ref, sb_ref, ctxn_ref, send_sems, recv_sems):
    my = lax.axis_index("i")
    left = lax.rem(my + N_DEV - 1, N_DEV)
    right = lax.rem(my + 1, N_DEV)

    barrier = pltpu.get_barrier_semaphore()
    for nbr in (left, right):
        pl.semaphore_signal(barrier, inc=1, device_id=(nbr,),
                            device_id_type=pl.DeviceIdType.MESH)
    pl.semaphore_wait(barrier, 2)

    qc_ref[0, 0] = q_ref[...]
    qc_ref[0, 1] = jnp.zeros((HQ, NJ, 64, DH), jnp.float32)
    sb_ref[0, :, :, :, 0:1] = jnp.full((HQ, NJ, 64, 1), -1e30, jnp.float32)
    sb_ref[0, :, :, :, 1:2] = jnp.zeros((HQ, NJ, 64, 1), jnp.float32)

    def hop_compute(slot):
        def t_body(t, carry):
            h = t // NJ
            j = t - h * NJ
            q = qc_ref[slot, 0, h, j]
            kt = kt_ref[h, j]
            v = v_ref[h, j]
            s = jnp.dot(q, kt, preferred_element_type=jnp.float32) * SCALE
            m_old = sb_ref[slot, h, j, :, 0:1]
            l_old = sb_ref[slot, h, j, :, 1:2]
            ctx_old = qc_ref[slot, 1, h, j]
            m_cur = jnp.max(s, axis=1, keepdims=True)
            m_new = jnp.maximum(m_old, m_cur)
            alpha = jnp.exp(m_old - m_new)
            p = jnp.exp(s - m_new)
            l_new = l_old * alpha + jnp.sum(p, axis=1, keepdims=True)
            ctx_new = ctx_old * alpha + jnp.dot(
                p, v, preferred_element_type=jnp.float32)
            qc_ref[slot, 1, h, j] = ctx_new
            sb_ref[slot, h, j, :, 0:1] = m_new
            sb_ref[slot, h, j, :, 1:2] = l_new
            return carry
        lax.fori_loop(0, HQ * NJ, t_body, 0)

    for hop in range(N_DEV):
        hop_compute(hop)
        rdma_qc = pltpu.make_async_remote_copy(
            src_ref=qc_ref.at[hop],
            dst_ref=qc_ref.at[hop + 1],
            send_sem=send_sems.at[0, hop],
            recv_sem=recv_sems.at[0, hop + 1],
            device_id=(right,),
            device_id_type=pl.DeviceIdType.MESH,
        )
        rdma_sb = pltpu.make_async_remote_copy(
            src_ref=sb_ref.at[hop],
            dst_ref=sb_ref.at[hop + 1],
            send_sem=send_sems.at[1, hop],
            recv_sem=recv_sems.at[1, hop + 1],
            device_id=(right,),
            device_id_type=pl.DeviceIdType.MESH,
        )
        rdma_qc.start()
        rdma_sb.start()
        rdma_qc.wait()
        rdma_sb.wait()

    for h in range(HQ):
        for j in range(NJ):
            ctx = qc_ref[N_DEV, 1, h, j]
            lsum = sb_ref[N_DEV, h, j, :, 1:2]
            ctxn_ref[j * 64:(j + 1) * 64, h * DH:(h + 1) * DH] = ctx / lsum
    out_ref[0] = jnp.dot(ctxn_ref[...], wo_ref[...],
                         preferred_element_type=jnp.float32)


def kernel(x, Wq, K_ext, V_ext, Wo):
    Q = jnp.dot(x[0], Wq, preferred_element_type=jnp.float32)
    Qr = Q.reshape(NJ, 64, HQ, DH).transpose(2, 0, 1, 3)
    K6 = K_ext[0].reshape(16, NJ, 64, HQ, DH)
    KT = K6.transpose(3, 1, 4, 0, 2).reshape(HQ, NJ, DH, KSEL)
    V6 = V_ext[0].reshape(16, NJ, 64, HQ, DH)
    Vr = V6.transpose(3, 1, 0, 2, 4).reshape(HQ, NJ, KSEL, DH)

    out = pl.pallas_call(
        _body,
        out_shape=jax.ShapeDtypeStruct((1, SQ, HQ * DH), jnp.float32),
        in_specs=[pl.BlockSpec(memory_space=pltpu.VMEM)] * 4,
        out_specs=pl.BlockSpec(memory_space=pltpu.VMEM),
        scratch_shapes=[
            pltpu.VMEM((N_DEV + 1, 2, HQ, NJ, 64, DH), jnp.float32),
            pltpu.VMEM((N_DEV + 1, HQ, NJ, 64, 2), jnp.float32),
            pltpu.VMEM((SQ, HQ * DH), jnp.float32),
            pltpu.SemaphoreType.DMA((2, N_DEV + 1)),
            pltpu.SemaphoreType.DMA((2, N_DEV + 1)),
        ],
        compiler_params=pltpu.CompilerParams(collective_id=0),
    )(Qr, KT, Vr, Wo)
    return out


# baseline (device time: 480316 ns/iter reference)
import jax
import jax.numpy as jnp
from jax import lax
from jax.experimental import pallas as pl
from jax.experimental.pallas import tpu as pltpu

N_DEV = 8
SQ = 256
HQ = 8
DH = 128
NJ = 4
KSEL = 1024
SCALE = 0.08838834764831843


NSLOT = 4


def _body(q_ref, kt_ref, v_ref, wo_ref, out_ref,
          qc_ref, sb_ref, ctxn_ref, send_sems, recv_sems, credit_sem):
    my = lax.axis_index("i")
    left = lax.rem(my + N_DEV - 1, N_DEV)
    right = lax.rem(my + 1, N_DEV)

    barrier = pltpu.get_barrier_semaphore()
    for nbr in (left, right):
        pl.semaphore_signal(barrier, inc=1, device_id=(nbr,),
                            device_id_type=pl.DeviceIdType.MESH)
    pl.semaphore_wait(barrier, 2)

    qc_ref[0, 0] = q_ref[...]
    qc_ref[0, 1] = jnp.zeros((HQ, NJ, 64, DH), jnp.float32)
    sb_ref[0, :, :, :, 0:1] = jnp.full((HQ, NJ, 64, 1), -1e30, jnp.float32)
    sb_ref[0, :, :, :, 1:2] = jnp.zeros((HQ, NJ, 64, 1), jnp.float32)

    def hop_compute(slot):
        def t_body(t, carry):
            h = t // NJ
            j = t - h * NJ
            q = qc_ref[slot, 0, h, j]
            kt = kt_ref[h, j]
            v = v_ref[h, j]
            s = jnp.dot(q, kt, preferred_element_type=jnp.float32) * SCALE
            m_old = sb_ref[slot, h, j, :, 0:1]
            l_old = sb_ref[slot, h, j, :, 1:2]
            ctx_old = qc_ref[slot, 1, h, j]
            m_cur = jnp.max(s, axis=1, keepdims=True)
            m_new = jnp.maximum(m_old, m_cur)
            alpha = jnp.exp(m_old - m_new)
            p = jnp.exp(s - m_new)
            l_new = l_old * alpha + jnp.sum(p, axis=1, keepdims=True)
            ctx_new = ctx_old * alpha + jnp.dot(
                p, v, preferred_element_type=jnp.float32)
            qc_ref[slot, 1, h, j] = ctx_new
            sb_ref[slot, h, j, :, 0:1] = m_new
            sb_ref[slot, h, j, :, 1:2] = l_new
            return carry
        lax.fori_loop(0, HQ * NJ, t_body, 0)

    for hop in range(N_DEV):
        s_cur = hop % NSLOT
        s_nxt = (hop + 1) % NSLOT
        hop_compute(s_cur)
        if hop >= NSLOT - 1:
            pl.semaphore_wait(credit_sem, 1)
        rdma_qc = pltpu.make_async_remote_copy(
            src_ref=qc_ref.at[s_cur],
            dst_ref=qc_ref.at[s_nxt],
            send_sem=send_sems.at[0, s_cur],
            recv_sem=recv_sems.at[0, s_nxt],
            device_id=(right,),
            device_id_type=pl.DeviceIdType.MESH,
        )
        rdma_sb = pltpu.make_async_remote_copy(
            src_ref=sb_ref.at[s_cur],
            dst_ref=sb_ref.at[s_nxt],
            send_sem=send_sems.at[1, s_cur],
            recv_sem=recv_sems.at[1, s_nxt],
            device_id=(right,),
            device_id_type=pl.DeviceIdType.MESH,
        )
        rdma_qc.start()
        rdma_sb.start()
        rdma_qc.wait()
        rdma_sb.wait()
        if hop < N_DEV - NSLOT + 1:
            pl.semaphore_signal(credit_sem, inc=1, device_id=(left,),
                                device_id_type=pl.DeviceIdType.MESH)

    home = N_DEV % NSLOT
    for h in range(HQ):
        for j in range(NJ):
            ctx = qc_ref[home, 1, h, j]
            lsum = sb_ref[home, h, j, :, 1:2]
            ctxn_ref[j * 64:(j + 1) * 64, h * DH:(h + 1) * DH] = ctx / lsum
    out_ref[0] = jnp.dot(ctxn_ref[...], wo_ref[...],
                         preferred_element_type=jnp.float32)


def kernel(x, Wq, K_ext, V_ext, Wo):
    Q = jnp.dot(x[0], Wq, preferred_element_type=jnp.float32)
    Qr = Q.reshape(NJ, 64, HQ, DH).transpose(2, 0, 1, 3)
    K6 = K_ext[0].reshape(16, NJ, 64, HQ, DH)
    KT = K6.transpose(3, 1, 4, 0, 2).reshape(HQ, NJ, DH, KSEL)
    V6 = V_ext[0].reshape(16, NJ, 64, HQ, DH)
    Vr = V6.transpose(3, 1, 0, 2, 4).reshape(HQ, NJ, KSEL, DH)

    out = pl.pallas_call(
        _body,
        out_shape=jax.ShapeDtypeStruct((1, SQ, HQ * DH), jnp.float32),
        in_specs=[pl.BlockSpec(memory_space=pltpu.VMEM)] * 4,
        out_specs=pl.BlockSpec(memory_space=pltpu.VMEM),
        scratch_shapes=[
            pltpu.VMEM((NSLOT, 2, HQ, NJ, 64, DH), jnp.float32),
            pltpu.VMEM((NSLOT, HQ, NJ, 64, 2), jnp.float32),
            pltpu.VMEM((SQ, HQ * DH), jnp.float32),
            pltpu.SemaphoreType.DMA((2, NSLOT)),
            pltpu.SemaphoreType.DMA((2, NSLOT)),
            pltpu.SemaphoreType.REGULAR,
        ],
        compiler_params=pltpu.CompilerParams(
            collective_id=0, vmem_limit_bytes=100 * 1024 * 1024),
    )(Qr, KT, Vr, Wo)
    return out


# device time: 412011 ns/iter; 1.1658x vs baseline; 1.1658x over previous
import jax
import jax.numpy as jnp
from jax import lax
from jax.experimental import pallas as pl
from jax.experimental.pallas import tpu as pltpu

N_DEV = 8
SQ = 256
HQ = 8
DH = 128
NJ = 4
KSEL = 1024
SCALE = 0.08838834764831843
NSLOT = 4


def _body(q_ref, kt_ref, v_ref, wo_ref, out_ref,
          qb_ref, cb_ref, sb_ref, ctxn_ref, send_sems, recv_sems,
          credit_sem):
    my = lax.axis_index("i")
    left = lax.rem(my + N_DEV - 1, N_DEV)
    right = lax.rem(my + 1, N_DEV)

    barrier = pltpu.get_barrier_semaphore()
    for nbr in (left, right):
        pl.semaphore_signal(barrier, inc=1, device_id=(nbr,),
                            device_id_type=pl.DeviceIdType.MESH)
    pl.semaphore_wait(barrier, 2)

    qb_ref[0] = q_ref[...]
    cb_ref[0] = jnp.zeros((HQ, NJ, 64, DH), jnp.float32)
    sb_ref[0, :, :, :, 0:1] = jnp.full((HQ, NJ, 64, 1), -1e30, jnp.float32)
    sb_ref[0, :, :, :, 1:2] = jnp.zeros((HQ, NJ, 64, 1), jnp.float32)

    def hop_compute(slot):
        def t_body(t, carry):
            h = t // NJ
            j = t - h * NJ
            q = qb_ref[slot, h, j]
            kt = kt_ref[h, j]
            v = v_ref[h, j]
            s = jnp.dot(q, kt, preferred_element_type=jnp.float32) * SCALE
            m_old = sb_ref[slot, h, j, :, 0:1]
            l_old = sb_ref[slot, h, j, :, 1:2]
            ctx_old = cb_ref[slot, h, j]
            m_cur = jnp.max(s, axis=1, keepdims=True)
            m_new = jnp.maximum(m_old, m_cur)
            alpha = jnp.exp(m_old - m_new)
            p = jnp.exp(s - m_new)
            l_new = l_old * alpha + jnp.sum(p, axis=1, keepdims=True)
            ctx_new = ctx_old * alpha + jnp.dot(
                p.astype(jnp.bfloat16), v, preferred_element_type=jnp.float32)
            cb_ref[slot, h, j] = ctx_new
            sb_ref[slot, h, j, :, 0:1] = m_new
            sb_ref[slot, h, j, :, 1:2] = l_new
            return carry
        lax.fori_loop(0, HQ * NJ, t_body, 0)

    def mk(buf, row, s_cur, s_nxt):
        return pltpu.make_async_remote_copy(
            src_ref=buf.at[s_cur],
            dst_ref=buf.at[s_nxt],
            send_sem=send_sems.at[row, s_cur],
            recv_sem=recv_sems.at[row, s_nxt],
            device_id=(right,),
            device_id_type=pl.DeviceIdType.MESH,
        )

    for hop in range(N_DEV):
        s_cur = hop % NSLOT
        s_nxt = (hop + 1) % NSLOT
        hop_compute(s_cur)
        if hop >= NSLOT - 1:
            pl.semaphore_wait(credit_sem, 1)
        rdmas = []
        if hop < N_DEV - 1:
            rdmas.append(mk(qb_ref, 0, s_cur, s_nxt))
        rdmas.append(mk(cb_ref, 1, s_cur, s_nxt))
        rdmas.append(mk(sb_ref, 2, s_cur, s_nxt))
        for r in rdmas:
            r.start()
        for r in rdmas:
            r.wait()
        if hop < N_DEV - NSLOT + 1:
            pl.semaphore_signal(credit_sem, inc=1, device_id=(left,),
                                device_id_type=pl.DeviceIdType.MESH)

    home = N_DEV % NSLOT
    for h in range(HQ):
        for j in range(NJ):
            ctx = cb_ref[home, h, j]
            lsum = sb_ref[home, h, j, :, 1:2]
            ctxn_ref[j * 64:(j + 1) * 64, h * DH:(h + 1) * DH] = ctx / lsum
    out_ref[0] = jnp.dot(ctxn_ref[...], wo_ref[...],
                         preferred_element_type=jnp.float32)


def kernel(x, Wq, K_ext, V_ext, Wo):
    Q = jnp.dot(x[0], Wq, preferred_element_type=jnp.float32)
    Qr = Q.reshape(NJ, 64, HQ, DH).transpose(2, 0, 1, 3)
    Qr = Qr.astype(jnp.bfloat16)
    K6 = K_ext[0].reshape(16, NJ, 64, HQ, DH)
    KT = K6.transpose(3, 1, 4, 0, 2).reshape(HQ, NJ, DH, KSEL)
    V6 = V_ext[0].reshape(16, NJ, 64, HQ, DH)
    Vr = V6.transpose(3, 1, 0, 2, 4).reshape(HQ, NJ, KSEL, DH)

    out = pl.pallas_call(
        _body,
        out_shape=jax.ShapeDtypeStruct((1, SQ, HQ * DH), jnp.float32),
        in_specs=[pl.BlockSpec(memory_space=pltpu.VMEM)] * 4,
        out_specs=pl.BlockSpec(memory_space=pltpu.VMEM),
        scratch_shapes=[
            pltpu.VMEM((NSLOT, HQ, NJ, 64, DH), jnp.bfloat16),
            pltpu.VMEM((NSLOT, HQ, NJ, 64, DH), jnp.float32),
            pltpu.VMEM((NSLOT, HQ, NJ, 64, 2), jnp.float32),
            pltpu.VMEM((SQ, HQ * DH), jnp.float32),
            pltpu.SemaphoreType.DMA((3, NSLOT)),
            pltpu.SemaphoreType.DMA((3, NSLOT)),
            pltpu.SemaphoreType.REGULAR,
        ],
        compiler_params=pltpu.CompilerParams(
            collective_id=0, vmem_limit_bytes=100 * 1024 * 1024),
    )(Qr, KT.astype(jnp.bfloat16), Vr.astype(jnp.bfloat16), Wo)
    return out
